# Initial kernel scaffold; baseline (speedup 1.0000x reference)
import jax
import jax.numpy as jnp
from jax.experimental import pallas as pl


def kernel(token_ids, table):
    raise NotImplementedError("write your pallas kernel here")


# speedup vs baseline: 1.7620x; 1.7620x over previous
import functools

import jax
import jax.numpy as jnp
from jax import lax
from jax.experimental import pallas as pl
from jax.experimental.pallas import tpu as pltpu
from jax.experimental.pallas import tpu_sc as plsc

_NW = 32
_W = 16
_NBUF = 4


@functools.lru_cache(maxsize=None)
def _make_gather(B, V, D):
    b_per_w = B // _NW
    ch = b_per_w // _W
    assert ch % _NBUF == 0
    mesh = plsc.VectorSubcoreMesh(core_axis_name="c", subcore_axis_name="s")

    scratch = [pltpu.VMEM((ch, _W), jnp.int32)]
    scratch += [pltpu.VMEM((_W, D), jnp.float32) for _ in range(_NBUF)]
    scratch += [pltpu.SemaphoreType.DMA for _ in range(2 * _NBUF)]

    @functools.partial(
        pl.kernel,
        mesh=mesh,
        out_type=jax.ShapeDtypeStruct((B, D), jnp.float32),
        scratch_types=scratch,
    )
    def k(ids_hbm, table_hbm, out_hbm, idx_v, *rest):
        bufs = list(rest[:_NBUF])
        gsem = list(rest[_NBUF:2 * _NBUF])
        ssem = list(rest[2 * _NBUF:])
        wid = lax.axis_index("s") * 2 + lax.axis_index("c")
        base = wid * b_per_w

        pltpu.sync_copy(ids_hbm.at[wid], idx_v)

        def g_start(c, b):
            pltpu.make_async_copy(table_hbm.at[idx_v.at[c]], bufs[b], gsem[b]).start()

        def g_wait(c, b):
            pltpu.make_async_copy(table_hbm.at[idx_v.at[c]], bufs[b], gsem[b]).wait()

        def s_start(c, b):
            pltpu.make_async_copy(
                bufs[b], out_hbm.at[pl.ds(base + c * _W, _W)], ssem[b]).start()

        def s_wait(c, b):
            pltpu.make_async_copy(
                bufs[b], out_hbm.at[pl.ds(base + c * _W, _W)], ssem[b]).wait()

        for b in range(_NBUF):
            g_start(b, b)

        def body(g, carry):
            c0 = g * _NBUF
            for b in range(_NBUF):
                c = c0 + b
                g_wait(c, b)
                s_start(c, b)
                s_wait(c, b)
                g_start(c + _NBUF, b)
            return carry

        lax.fori_loop(0, ch // _NBUF - 1, body, 0)

        c0 = ch - _NBUF
        for b in range(_NBUF):
            c = c0 + b
            g_wait(c, b)
            s_start(c, b)
            s_wait(c, b)

    return k


def kernel(token_ids, table):
    B0, B1 = token_ids.shape
    B = B0 * B1
    V, D = table.shape
    ids = token_ids.reshape(_NW, (B // _NW) // _W, _W)
    out = _make_gather(B, V, D)(ids, table)
    return out.reshape(B0, B1, D)
